# Initial kernel scaffold; baseline (speedup 1.0000x reference)
import jax
import jax.numpy as jnp
from jax.experimental import pallas as pl


def kernel(inputs, table):
    raise NotImplementedError("write your pallas kernel here")


# speedup vs baseline: 1.0932x; 1.0932x over previous
import functools
import jax
import jax.numpy as jnp
from jax import lax
from jax.experimental import pallas as pl
from jax.experimental.pallas import tpu as pltpu
from jax.experimental.pallas import tpu_sc as plsc

BATCH = 16384
HIST = 50
EMBED = 32
N = BATCH * HIST
NC, NS = 2, 16
NW = NC * NS
B_PER_W = N // NW
CHUNK = 1024
NCHUNK = B_PER_W // CHUNK

_mesh = plsc.VectorSubcoreMesh(
    core_axis_name="c", subcore_axis_name="s", num_cores=NC, num_subcores=NS
)


@functools.partial(
    pl.kernel,
    out_type=jax.ShapeDtypeStruct((N, EMBED), jnp.float32),
    mesh=_mesh,
    scratch_types=[
        pltpu.VMEM((CHUNK,), jnp.int32),
        pltpu.VMEM((CHUNK, EMBED), jnp.float32),
        pltpu.SemaphoreType.DMA,
    ],
    compiler_params=pltpu.CompilerParams(use_tc_tiling_on_sc=False),
)
def _gather(table_hbm, idx_hbm, out_hbm, idx_v, rows_v, sem):
    wid = lax.axis_index("s") * NC + lax.axis_index("c")
    base = wid * B_PER_W

    def body(i, _):
        off = base + i * CHUNK
        pltpu.sync_copy(idx_hbm.at[pl.ds(off, CHUNK)], idx_v)
        pltpu.async_copy(table_hbm.at[idx_v], rows_v, sem).wait()
        pltpu.sync_copy(rows_v, out_hbm.at[pl.ds(off, CHUNK)])
        return ()

    lax.fori_loop(0, NCHUNK, body, ())


def kernel(inputs, table):
    idx = inputs.reshape(N).astype(jnp.int32)
    out = _gather(table, idx)
    return out.reshape(BATCH, HIST, EMBED)


# speedup vs baseline: 1.1119x; 1.0171x over previous
import functools
import jax
import jax.numpy as jnp
from jax import lax
from jax.experimental import pallas as pl
from jax.experimental.pallas import tpu as pltpu
from jax.experimental.pallas import tpu_sc as plsc

BATCH = 16384
HIST = 50
EMBED = 32
N = BATCH * HIST
NC, NS = 2, 16
NW = NC * NS
B_PER_W = N // NW
CHUNK = 1024
NCHUNK = B_PER_W // CHUNK

_mesh = plsc.VectorSubcoreMesh(
    core_axis_name="c", subcore_axis_name="s", num_cores=NC, num_subcores=NS
)


@functools.partial(
    pl.kernel,
    out_type=jax.ShapeDtypeStruct((N, EMBED), jnp.float32),
    mesh=_mesh,
    scratch_types=[
        pltpu.VMEM((NCHUNK, CHUNK), jnp.int32),
        pltpu.VMEM((CHUNK, EMBED), jnp.float32),
        pltpu.VMEM((CHUNK, EMBED), jnp.float32),
        pltpu.SemaphoreType.DMA,
        pltpu.SemaphoreType.DMA,
        pltpu.SemaphoreType.DMA,
        pltpu.SemaphoreType.DMA,
    ],
    compiler_params=pltpu.CompilerParams(use_tc_tiling_on_sc=False),
)
def _gather(table_hbm, idx_hbm, out_hbm, idx_v, rows0, rows1, g0, g1, s0, s1):
    wid = lax.axis_index("s") * NC + lax.axis_index("c")
    base = wid * B_PER_W
    rows = (rows0, rows1)
    gsem = (g0, g1)
    ssem = (s0, s1)

    pltpu.sync_copy(idx_hbm.at[wid], idx_v)

    def start_gather(i):
        return pltpu.async_copy(table_hbm.at[idx_v.at[i]], rows[i % 2], gsem[i % 2])

    def start_store(i):
        return pltpu.async_copy(
            rows[i % 2], out_hbm.at[pl.ds(base + i * CHUNK, CHUNK)], ssem[i % 2]
        )

    gathers = [None] * NCHUNK
    stores = [None] * NCHUNK
    gathers[0] = start_gather(0)
    for i in range(NCHUNK):
        if i + 1 < NCHUNK:
            if i >= 1:
                stores[i - 1].wait()
            gathers[i + 1] = start_gather(i + 1)
        gathers[i].wait()
        stores[i] = start_store(i)
    stores[NCHUNK - 2].wait()
    stores[NCHUNK - 1].wait()


def kernel(inputs, table):
    idx = inputs.reshape(NW, NCHUNK, CHUNK).astype(jnp.int32)
    out = _gather(table, idx)
    return out.reshape(BATCH, HIST, EMBED)


# speedup vs baseline: 1.7678x; 1.5899x over previous
import functools
import jax
import jax.numpy as jnp
from jax import lax
from jax.experimental import pallas as pl
from jax.experimental.pallas import tpu as pltpu
from jax.experimental.pallas import tpu_sc as plsc

BATCH = 16384
HIST = 50
EMBED = 32
NC, NS = 2, 16
NW = NC * NS
R_PER_W = BATCH // NW
CHUNK = 16
NCHUNK = R_PER_W // CHUNK

_mesh = plsc.VectorSubcoreMesh(
    core_axis_name="c", subcore_axis_name="s", num_cores=NC, num_subcores=NS
)


@functools.partial(
    pl.kernel,
    out_type=jax.ShapeDtypeStruct((BATCH, HIST, EMBED), jnp.float32),
    mesh=_mesh,
    scratch_types=[
        pltpu.VMEM((R_PER_W, HIST), jnp.int32),
        pltpu.VMEM((CHUNK, HIST, EMBED), jnp.float32),
        pltpu.SemaphoreType.DMA,
        pltpu.SemaphoreType.DMA,
    ],
    compiler_params=pltpu.CompilerParams(use_tc_tiling_on_sc=False),
)
def _gather(table_hbm, idx_hbm, out_hbm, idx_v, rows_v, gsem, ssem):
    wid = lax.axis_index("s") * NC + lax.axis_index("c")
    base = wid * R_PER_W

    pltpu.sync_copy(idx_hbm.at[pl.ds(base, R_PER_W)], idx_v)

    def body(i, _):
        for j in range(CHUNK):
            pltpu.async_copy(
                table_hbm.at[idx_v.at[i * CHUNK + j]],
                rows_v.at[j],
                gsem,
            )
        for j in range(CHUNK):
            pltpu.make_async_copy(
                table_hbm.at[idx_v.at[i * CHUNK + j]],
                rows_v.at[j],
                gsem,
            ).wait()
        pltpu.async_copy(rows_v, out_hbm.at[pl.ds(base + i * CHUNK, CHUNK)], ssem).wait()
        return ()

    lax.fori_loop(0, NCHUNK, body, ())


def kernel(inputs, table):
    return _gather(table, inputs)
